# baseline (speedup 1.0000x reference)
import functools

import jax
import jax.numpy as jnp
from jax import lax
from jax.experimental import pallas as pl
from jax.experimental.pallas import tpu as pltpu
from jax.experimental.pallas import tpu_sc as plsc

N = 10000
E = 320000
D = 128
H = D // 2
G = 32
NC, NS, L = 2, 16, 16
K = 128
EPT = 20480
CHUNKS = EPT // K
DCHUNKS = CHUNKS // NC
EPAD = NS * EPT
NPAD = 10240
RPT = NPAD // NS
PADDST = N

_mesh = plsc.VectorSubcoreMesh(core_axis_name="c", subcore_axis_name="s")
_f32 = jnp.float32


@functools.partial(
    pl.kernel,
    out_type=jax.ShapeDtypeStruct((NC, NPAD, L), _f32),
    mesh=_mesh,
    scratch_types=[
        pltpu.VMEM((DCHUNKS, K), jnp.int32),
        pltpu.VMEM((K, L), _f32),
        pltpu.VMEM((K, L), _f32),
        pltpu.VMEM_SHARED((NPAD, L), _f32),
    ],
    compiler_params=pltpu.CompilerParams(use_tc_tiling_on_sc=False),
)
def _deg_sc(dst_hbm, out_hbm, didx, oh_buf, zbuf, acc):
    c = lax.axis_index("c")
    s = lax.axis_index("s")
    oh = jnp.where(lax.iota(jnp.int32, L) == 0, _f32(1.0), _f32(0.0))
    z16 = jnp.zeros((L,), _f32)

    @pl.loop(0, K)
    def _(j):
        oh_buf[j, pl.ds(0, L)] = oh
        zbuf[j, pl.ds(0, L)] = z16

    @pl.loop(0, RPT // K)
    def _(j):
        pltpu.sync_copy(zbuf, acc.at[pl.ds(s * RPT + j * K, K)])

    plsc.subcore_barrier()
    pltpu.sync_copy(dst_hbm.at[c, s], didx)

    @pl.loop(0, DCHUNKS)
    def _(j):
        pltpu.sync_copy(oh_buf, acc.at[didx.at[j]], add=True)

    plsc.subcore_barrier()
    pltpu.sync_copy(acc.at[pl.ds(s * RPT, RPT)], out_hbm.at[c, pl.ds(s * RPT, RPT)])


_NBUF = 5
assert CHUNKS % _NBUF == 0


@functools.partial(
    pl.kernel,
    out_type=jax.ShapeDtypeStruct((NC, NPAD, H), _f32),
    mesh=_mesh,
    scratch_types=[
        pltpu.VMEM((CHUNKS, K), jnp.int32),
        pltpu.VMEM((CHUNKS, K), jnp.int32),
        [pltpu.VMEM((K, H), _f32)] * _NBUF,
        pltpu.VMEM_SHARED((NPAD, H), _f32),
        [pltpu.SemaphoreType.DMA] * _NBUF,
        [pltpu.SemaphoreType.DMA] * _NBUF,
    ],
    compiler_params=pltpu.CompilerParams(use_tc_tiling_on_sc=False),
)
def _agg_sc(y_hbm, src_hbm, dst_hbm, out_hbm, sidx, didx, rows, acc,
            gsem, ssem):
    c = lax.axis_index("c")
    s = lax.axis_index("s")
    z16 = jnp.zeros((L,), _f32)

    zbuf = rows[0]

    @pl.loop(0, K)
    def _(j):
        @pl.loop(0, H, step=L)
        def _(k2):
            zbuf[j, pl.ds(k2, L)] = z16

    @pl.loop(0, RPT // K)
    def _(j):
        pltpu.sync_copy(zbuf, acc.at[pl.ds(s * RPT + j * K, K)])

    plsc.subcore_barrier()
    pltpu.sync_copy(src_hbm.at[c, s], sidx)
    pltpu.sync_copy(dst_hbm.at[s], didx)

    def g_start(j, b):
        pltpu.async_copy(y_hbm.at[sidx.at[j]], rows[b], gsem[b])

    def g_wait(j, b):
        pltpu.make_async_copy(y_hbm.at[sidx.at[j]], rows[b], gsem[b]).wait()

    def s_start(j, b):
        pltpu.async_copy(rows[b], acc.at[didx.at[j]], ssem[b], add=True)

    def s_wait(j, b):
        pltpu.make_async_copy(rows[b], acc.at[didx.at[j]], ssem[b]).wait()

    for b in range(_NBUF):
        g_start(b, b)

    @pl.loop(0, CHUNKS // _NBUF - 1)
    def _(i):
        j0 = i * _NBUF
        for b in range(_NBUF):
            g_wait(j0 + b, b)
            s_start(j0 + b, b)
        for b in range(_NBUF):
            s_wait(j0 + b, b)
            g_start(j0 + _NBUF + b, b)

    j0 = CHUNKS - _NBUF
    for b in range(_NBUF):
        g_wait(j0 + b, b)
        s_start(j0 + b, b)
    for b in range(_NBUF):
        s_wait(j0 + b, b)

    plsc.subcore_barrier()
    pltpu.sync_copy(acc.at[pl.ds(s * RPT, RPT)], out_hbm.at[c, pl.ds(s * RPT, RPT)])


_BLK = 1000
_GRID = N // _BLK


def _row_spec(w):
    return pl.BlockSpec((_BLK, w), lambda i: (i, 0))


def _half_spec():
    return pl.BlockSpec((2, _BLK, H), lambda i: (0, i, 0))


def _full_spec(h, w):
    return pl.BlockSpec((h, w), lambda i: (0, 0))


def _split(r):
    return jnp.stack([r[:, :H], r[:, H:]], axis=0)


def _k0_body(x, w0, m0):
    m0[...] = jnp.dot(x[...], w0[...], preferred_element_type=_f32)


def _tc_k0(x, w0):
    return pl.pallas_call(
        _k0_body,
        grid=(_GRID,),
        in_specs=[_row_spec(D), _full_spec(D, D)],
        out_specs=_row_spec(D),
        out_shape=jax.ShapeDtypeStruct((N, D), _f32),
    )(x, w0)


def _k1_body(dg0, dg1, m0, y0, dinv):
    deg = dg0[:, 0:1] + dg1[:, 0:1] + _f32(1.0)
    di = lax.rsqrt(jnp.maximum(deg, _f32(1.0)))
    dinv[...] = di
    y0[...] = _split(m0[...] * di)


def _tc_k1(dg0, dg1, m0):
    return pl.pallas_call(
        _k1_body,
        grid=(_GRID,),
        in_specs=[_row_spec(L), _row_spec(L), _row_spec(D)],
        out_specs=[_half_spec(), _row_spec(1)],
        out_shape=[jax.ShapeDtypeStruct((2, N, H), _f32),
                   jax.ShapeDtypeStruct((N, 1), _f32)],
    )(dg0, dg1, m0)


def _merge(p, y):
    agg = jnp.concatenate([p[0], p[1]], axis=1)
    yy = jnp.concatenate([y[0], y[1]], axis=1)
    return agg + yy


def _k2_body(p, y, dinv, b, wn, out):
    h = jax.nn.relu(_merge(p[...], y[...]) * dinv[...] + b[...])
    out[...] = _split(jnp.dot(h, wn[...], preferred_element_type=_f32) * dinv[...])


def _tc_k2(p, y, dinv, b, wn):
    return pl.pallas_call(
        _k2_body,
        grid=(_GRID,),
        in_specs=[_half_spec(), _half_spec(), _row_spec(1),
                  _full_spec(1, D), _full_spec(D, D)],
        out_specs=_half_spec(),
        out_shape=jax.ShapeDtypeStruct((2, N, H), _f32),
    )(p, y, dinv, b, wn)


def _k3_body(p, y, dinv, b, wf1, bf1, z, sums):
    h = jax.nn.relu(_merge(p[...], y[...]) * dinv[...] + b[...])
    zz = jnp.dot(h, wf1[...], preferred_element_type=_f32) + bf1[...]
    z[...] = zz

    @pl.when(pl.program_id(0) == 0)
    def _():
        sums[...] = jnp.zeros_like(sums)

    sums[0:1, :] += jnp.sum(zz, axis=0, keepdims=True)
    sums[1:2, :] += jnp.sum(zz * zz, axis=0, keepdims=True)


def _tc_k3(p, y, dinv, b, wf1, bf1):
    return pl.pallas_call(
        _k3_body,
        grid=(_GRID,),
        in_specs=[_half_spec(), _half_spec(), _row_spec(1),
                  _full_spec(1, D), _full_spec(D, D), _full_spec(1, D)],
        out_specs=[_row_spec(D), _full_spec(8, D)],
        out_shape=[jax.ShapeDtypeStruct((N, D), _f32),
                   jax.ShapeDtypeStruct((8, D), _f32)],
    )(p, y, dinv, b, wf1, bf1)


def _k4_body(z, scale, shift, wf2, bf2, bat, pmax):
    zn = jax.nn.relu(z[...] * scale[...] + shift[...])
    h3 = jax.nn.relu(jnp.dot(zn, wf2[...], preferred_element_type=_f32) + bf2[...])
    bb = bat[...]

    @pl.when(pl.program_id(0) == 0)
    def _():
        pmax[...] = jnp.full_like(pmax, -jnp.inf)

    for g in range(G):
        m = jnp.max(jnp.where(bb == g, h3, -jnp.inf), axis=0, keepdims=True)
        pmax[g:g + 1, :] = jnp.maximum(pmax[g:g + 1, :], m)


def _tc_k4(z, scale, shift, wf2, bf2, bat):
    return pl.pallas_call(
        _k4_body,
        grid=(_GRID,),
        in_specs=[_row_spec(D), _full_spec(1, D), _full_spec(1, D),
                  _full_spec(D, D), _full_spec(1, D), _row_spec(1)],
        out_specs=_full_spec(G, D),
        out_shape=jax.ShapeDtypeStruct((G, D), _f32),
    )(z, scale, shift, wf2, bf2, bat)


def _k5_body(p, wm1, bm1, gm, betam, wm2, bm2, out):
    p1 = jnp.dot(p[...], wm1[...], preferred_element_type=_f32) + bm1[...]
    mu = jnp.mean(p1, axis=0, keepdims=True)
    var = jnp.mean((p1 - mu) * (p1 - mu), axis=0, keepdims=True)
    p1 = (p1 - mu) * lax.rsqrt(var + _f32(1e-5)) * gm[...] + betam[...]
    p1 = jax.nn.relu(p1)
    out[...] = jnp.dot(p1, wm2[...], preferred_element_type=_f32) + bm2[...]


def _tc_k5(p, wm1, bm1, gm, betam, wm2, bm2):
    return pl.pallas_call(
        _k5_body,
        grid=(1,),
        in_specs=[_full_spec(G, D), _full_spec(D, H), _full_spec(1, H),
                  _full_spec(1, H), _full_spec(1, H), _full_spec(H, D),
                  _full_spec(1, D)],
        out_specs=_full_spec(G, D),
        out_shape=jax.ShapeDtypeStruct((G, D), _f32),
    )(p, wm1, bm1, gm, betam, wm2, bm2)


def kernel(x, edge_index, edge_attr, batch, W0, b0, W1, b1, Wf1, bf1, gf, betaf,
           Wf2, bf2, Wm1, bm1, gm, betam, Wm2, bm2):
    src = edge_index[0]
    dst = edge_index[1]
    srcpad = jnp.concatenate([src, jnp.zeros((EPAD - E,), jnp.int32)])
    srcp = jnp.stack([srcpad, srcpad + N]).reshape(NC, NS, CHUNKS, K)
    dstp = jnp.concatenate([dst, jnp.full((EPAD - E,), PADDST, jnp.int32)]
                           ).reshape(NS, CHUNKS, K)
    dstd = dstp.reshape(NS, NC, DCHUNKS, K).transpose(1, 0, 2, 3)

    m0 = _tc_k0(x, W0)
    degp = _deg_sc(dstd)
    y0, dinv = _tc_k1(degp[0, :N, :], degp[1, :N, :], m0)

    parts0 = _agg_sc(y0.reshape(NC * N, H), srcp, dstp)
    y1 = _tc_k2(parts0[:, :N, :], y0, dinv, b0.reshape(1, D), W1)

    parts1 = _agg_sc(y1.reshape(NC * N, H), srcp, dstp)
    z, sums = _tc_k3(parts1[:, :N, :], y1, dinv, b1.reshape(1, D),
                     Wf1, bf1.reshape(1, D))

    mu = sums[0:1, :] / N
    var = sums[1:2, :] / N - mu * mu
    scale = gf.reshape(1, D) * lax.rsqrt(var + 1e-5)
    shift = betaf.reshape(1, D) - mu * scale

    pmax = _tc_k4(z, scale, shift, Wf2, bf2.reshape(1, D), batch.reshape(N, 1))

    wm2p = jnp.pad(Wm2, ((0, 0), (0, D - 1)))
    bm2p = jnp.pad(bm2.reshape(1, 1), ((0, 0), (0, D - 1)))
    out = _tc_k5(pmax, Wm1, bm1.reshape(1, H), gm.reshape(1, H),
                 betam.reshape(1, H), wm2p, bm2p)
    return out[:, 0]


# speedup vs baseline: 1.1818x; 1.1818x over previous
import functools

import jax
import jax.numpy as jnp
from jax import lax
from jax.experimental import pallas as pl
from jax.experimental.pallas import tpu as pltpu
from jax.experimental.pallas import tpu_sc as plsc

N = 10000
E = 320000
D = 128
G = 32
NC, NS, L = 2, 16, 16
K = 128
EPT = 20480
CHUNKS = EPT // K
DCHUNKS = CHUNKS // NC
EPAD = NS * EPT
NPAD = 10240
RPT = NPAD // NS
PADDST = N
KA = 64
CHA = EPT // KA
HALFN = N // 2
AROWS = 5120
RPT2 = AROWS // NS
TRASH = HALFN
VPE = EPT // L

_mesh = plsc.VectorSubcoreMesh(core_axis_name="c", subcore_axis_name="s")
_f32 = jnp.float32
_i32 = jnp.int32


@functools.partial(
    pl.kernel,
    out_type=jax.ShapeDtypeStruct((NC, NPAD, L), _f32),
    mesh=_mesh,
    scratch_types=[
        pltpu.VMEM((DCHUNKS, K), _i32),
        pltpu.VMEM((K, L), _f32),
        pltpu.VMEM((K, L), _f32),
        pltpu.VMEM_SHARED((NPAD, L), _f32),
    ],
    compiler_params=pltpu.CompilerParams(use_tc_tiling_on_sc=False),
)
def _deg_sc(dst_hbm, out_hbm, didx, oh_buf, zbuf, acc):
    c = lax.axis_index("c")
    s = lax.axis_index("s")
    oh = jnp.where(lax.iota(_i32, L) == 0, _f32(1.0), _f32(0.0))
    z16 = jnp.zeros((L,), _f32)

    @pl.loop(0, K)
    def _(j):
        oh_buf[j, pl.ds(0, L)] = oh
        zbuf[j, pl.ds(0, L)] = z16

    @pl.loop(0, RPT // K)
    def _(j):
        pltpu.sync_copy(zbuf, acc.at[pl.ds(s * RPT + j * K, K)])

    plsc.subcore_barrier()
    pltpu.sync_copy(dst_hbm.at[c, s], didx)

    @pl.loop(0, DCHUNKS)
    def _(j):
        pltpu.sync_copy(oh_buf, acc.at[didx.at[j]], add=True)

    plsc.subcore_barrier()
    pltpu.sync_copy(acc.at[pl.ds(s * RPT, RPT)], out_hbm.at[c, pl.ds(s * RPT, RPT)])


@functools.partial(
    pl.kernel,
    out_type=[jax.ShapeDtypeStruct((NC, NS, EPT), _i32),
              jax.ShapeDtypeStruct((NC, NS, EPT), _i32),
              jax.ShapeDtypeStruct((NC, NS, L), _i32)],
    mesh=_mesh,
    scratch_types=[
        pltpu.VMEM((EPT,), _i32),
        pltpu.VMEM((EPT,), _i32),
        pltpu.VMEM((EPT,), _i32),
        pltpu.VMEM((EPT,), _i32),
        pltpu.VMEM((L,), _i32),
    ],
    compiler_params=pltpu.CompilerParams(use_tc_tiling_on_sc=False,
                                         needs_layout_passes=False),
)
def _part_sc(src_hbm, dst_hbm, so_hbm, do_hbm, cnt_hbm,
             sbuf, dbuf, sobuf, dobuf, cbuf):
    c = lax.axis_index("c")
    s = lax.axis_index("s")
    lo = c * HALFN
    pltpu.sync_copy(src_hbm.at[s], sbuf)
    pltpu.sync_copy(dst_hbm.at[s], dbuf)
    trash_s = jnp.zeros((L,), _i32)
    trash_d = jnp.full((L,), TRASH, _i32)

    @pl.loop(0, VPE)
    def _(v):
        sobuf[pl.ds(v * L, L)] = trash_s
        dobuf[pl.ds(v * L, L)] = trash_d

    @pl.loop(0, VPE, init_carry=jnp.int32(0))
    def cnt(v, cur):
        s16 = sbuf[pl.ds(v * L, L)]
        d16 = dbuf[pl.ds(v * L, L)]
        dl = d16 - lo
        m = (dl >= 0) & (dl < HALFN)
        plsc.store_compressed(sobuf.at[pl.ds(cur, L)], s16, mask=m)
        plsc.store_compressed(dobuf.at[pl.ds(cur, L)], dl, mask=m)
        return cur + jnp.sum(jnp.where(m, 1, 0).astype(_i32))

    pltpu.sync_copy(sobuf, so_hbm.at[c, s])
    pltpu.sync_copy(dobuf, do_hbm.at[c, s])
    cbuf[pl.ds(0, L)] = jnp.full((L,), cnt, _i32)
    pltpu.sync_copy(cbuf, cnt_hbm.at[c, s])


_NBUF = 4


@functools.partial(
    pl.kernel,
    out_type=jax.ShapeDtypeStruct((NC, AROWS, D), _f32),
    mesh=_mesh,
    scratch_types=[
        pltpu.VMEM((CHA, KA), _i32),
        pltpu.VMEM((CHA, KA), _i32),
        [pltpu.VMEM((KA, D), _f32)] * _NBUF,
        pltpu.VMEM((L,), _i32),
        pltpu.VMEM_SHARED((AROWS, D), _f32),
        [pltpu.SemaphoreType.DMA] * _NBUF,
        [pltpu.SemaphoreType.DMA] * _NBUF,
    ],
    compiler_params=pltpu.CompilerParams(use_tc_tiling_on_sc=False,
                                         needs_layout_passes=False),
)
def _agg_sc(y_hbm, src_hbm, dst_hbm, cnt_hbm, out_hbm,
            sidx, didx, rows, cbuf, acc, gsem, ssem):
    c = lax.axis_index("c")
    s = lax.axis_index("s")
    z16 = jnp.zeros((L,), _f32)

    zbuf = rows[0]

    @pl.loop(0, KA)
    def _(j):
        @pl.loop(0, D, step=L)
        def _(k2):
            zbuf[j, pl.ds(k2, L)] = z16

    @pl.loop(0, RPT2 // KA)
    def _(j):
        pltpu.sync_copy(zbuf, acc.at[pl.ds(s * RPT2 + j * KA, KA)])

    plsc.subcore_barrier()
    pltpu.sync_copy(src_hbm.at[c, s], sidx)
    pltpu.sync_copy(dst_hbm.at[c, s], didx)
    pltpu.sync_copy(cnt_hbm.at[c, s], cbuf)
    cnt = jnp.max(cbuf[...])
    nch = (cnt + (KA - 1)) // KA
    nrounds = jnp.maximum((nch + (_NBUF - 1)) // _NBUF, 1)

    def g_start(j, b):
        pltpu.async_copy(y_hbm.at[sidx.at[j]], rows[b], gsem[b])

    def g_wait(j, b):
        pltpu.make_async_copy(y_hbm.at[sidx.at[j]], rows[b], gsem[b]).wait()

    def s_start(j, b):
        pltpu.async_copy(rows[b], acc.at[didx.at[j]], ssem[b], add=True)

    def s_wait(j, b):
        pltpu.make_async_copy(rows[b], acc.at[didx.at[j]], ssem[b]).wait()

    for b in range(_NBUF):
        g_start(b, b)

    @pl.loop(0, nrounds - 1)
    def _(i):
        j0 = i * _NBUF
        for b in range(_NBUF):
            g_wait(j0 + b, b)
            s_start(j0 + b, b)
        for b in range(_NBUF):
            s_wait(j0 + b, b)
            g_start(j0 + _NBUF + b, b)

    j0 = (nrounds - 1) * _NBUF
    for b in range(_NBUF):
        g_wait(j0 + b, b)
        s_start(j0 + b, b)
    for b in range(_NBUF):
        s_wait(j0 + b, b)

    plsc.subcore_barrier()
    pltpu.sync_copy(acc.at[pl.ds(s * RPT2, RPT2)],
                    out_hbm.at[c, pl.ds(s * RPT2, RPT2)])


_BLK = 1000
_GRID = N // _BLK
_PB = HALFN // _BLK


def _row_spec(w):
    return pl.BlockSpec((_BLK, w), lambda i: (i, 0))


def _part_spec():
    return pl.BlockSpec((1, _BLK, D), lambda i: (i // _PB, i % _PB, 0))


def _full_spec(h, w):
    return pl.BlockSpec((h, w), lambda i: (0, 0))


def _k0_body(x, w0, m0):
    m0[...] = jnp.dot(x[...], w0[...], preferred_element_type=_f32)


def _tc_k0(x, w0):
    return pl.pallas_call(
        _k0_body,
        grid=(_GRID,),
        in_specs=[_row_spec(D), _full_spec(D, D)],
        out_specs=_row_spec(D),
        out_shape=jax.ShapeDtypeStruct((N, D), _f32),
    )(x, w0)


def _k1_body(dg0, dg1, m0, y0, dinv):
    deg = dg0[:, 0:1] + dg1[:, 0:1] + _f32(1.0)
    di = lax.rsqrt(jnp.maximum(deg, _f32(1.0)))
    dinv[...] = di
    y0[...] = m0[...] * di


def _tc_k1(dg0, dg1, m0):
    return pl.pallas_call(
        _k1_body,
        grid=(_GRID,),
        in_specs=[_row_spec(L), _row_spec(L), _row_spec(D)],
        out_specs=[_row_spec(D), _row_spec(1)],
        out_shape=[jax.ShapeDtypeStruct((N, D), _f32),
                   jax.ShapeDtypeStruct((N, 1), _f32)],
    )(dg0, dg1, m0)


def _k2_body(p, y, dinv, b, wn, out):
    h = jax.nn.relu((p[0] + y[...]) * dinv[...] + b[...])
    out[...] = jnp.dot(h, wn[...], preferred_element_type=_f32) * dinv[...]


def _tc_k2(p, y, dinv, b, wn):
    return pl.pallas_call(
        _k2_body,
        grid=(_GRID,),
        in_specs=[_part_spec(), _row_spec(D), _row_spec(1),
                  _full_spec(1, D), _full_spec(D, D)],
        out_specs=_row_spec(D),
        out_shape=jax.ShapeDtypeStruct((N, D), _f32),
    )(p, y, dinv, b, wn)


def _k3_body(p, y, dinv, b, wf1, bf1, z, sums):
    h = jax.nn.relu((p[0] + y[...]) * dinv[...] + b[...])
    zz = jnp.dot(h, wf1[...], preferred_element_type=_f32) + bf1[...]
    z[...] = zz

    @pl.when(pl.program_id(0) == 0)
    def _():
        sums[...] = jnp.zeros_like(sums)

    sums[0:1, :] += jnp.sum(zz, axis=0, keepdims=True)
    sums[1:2, :] += jnp.sum(zz * zz, axis=0, keepdims=True)


def _tc_k3(p, y, dinv, b, wf1, bf1):
    return pl.pallas_call(
        _k3_body,
        grid=(_GRID,),
        in_specs=[_part_spec(), _row_spec(D), _row_spec(1),
                  _full_spec(1, D), _full_spec(D, D), _full_spec(1, D)],
        out_specs=[_row_spec(D), _full_spec(8, D)],
        out_shape=[jax.ShapeDtypeStruct((N, D), _f32),
                   jax.ShapeDtypeStruct((8, D), _f32)],
    )(p, y, dinv, b, wf1, bf1)


def _k4_body(z, scale, shift, wf2, bf2, bat, pmax):
    zn = jax.nn.relu(z[...] * scale[...] + shift[...])
    h3 = jax.nn.relu(jnp.dot(zn, wf2[...], preferred_element_type=_f32) + bf2[...])
    bb = bat[...]

    @pl.when(pl.program_id(0) == 0)
    def _():
        pmax[...] = jnp.full_like(pmax, -jnp.inf)

    for g in range(G):
        m = jnp.max(jnp.where(bb == g, h3, -jnp.inf), axis=0, keepdims=True)
        pmax[g:g + 1, :] = jnp.maximum(pmax[g:g + 1, :], m)


def _tc_k4(z, scale, shift, wf2, bf2, bat):
    return pl.pallas_call(
        _k4_body,
        grid=(_GRID,),
        in_specs=[_row_spec(D), _full_spec(1, D), _full_spec(1, D),
                  _full_spec(D, D), _full_spec(1, D), _row_spec(1)],
        out_specs=_full_spec(G, D),
        out_shape=jax.ShapeDtypeStruct((G, D), _f32),
    )(z, scale, shift, wf2, bf2, bat)


def _k5_body(p, wm1, bm1, gm, betam, wm2, bm2, out):
    p1 = jnp.dot(p[...], wm1[...], preferred_element_type=_f32) + bm1[...]
    mu = jnp.mean(p1, axis=0, keepdims=True)
    var = jnp.mean((p1 - mu) * (p1 - mu), axis=0, keepdims=True)
    p1 = (p1 - mu) * lax.rsqrt(var + _f32(1e-5)) * gm[...] + betam[...]
    p1 = jax.nn.relu(p1)
    out[...] = jnp.dot(p1, wm2[...], preferred_element_type=_f32) + bm2[...]


def _tc_k5(p, wm1, bm1, gm, betam, wm2, bm2):
    HW = D // 2
    return pl.pallas_call(
        _k5_body,
        grid=(1,),
        in_specs=[_full_spec(G, D), _full_spec(D, HW), _full_spec(1, HW),
                  _full_spec(1, HW), _full_spec(1, HW), _full_spec(HW, D),
                  _full_spec(1, D)],
        out_specs=_full_spec(G, D),
        out_shape=jax.ShapeDtypeStruct((G, D), _f32),
    )(p, wm1, bm1, gm, betam, wm2, bm2)


def kernel(x, edge_index, edge_attr, batch, W0, b0, W1, b1, Wf1, bf1, gf, betaf,
           Wf2, bf2, Wm1, bm1, gm, betam, Wm2, bm2):
    src = edge_index[0]
    dst = edge_index[1]
    srcpad = jnp.concatenate([src, jnp.zeros((EPAD - E,), _i32)]).reshape(NS, EPT)
    dstpad = jnp.concatenate([dst, jnp.full((EPAD - E,), PADDST, _i32)]
                             ).reshape(NS, EPT)
    dstd = dstpad.reshape(NS, NC, DCHUNKS, K).transpose(1, 0, 2, 3)

    so, do_, cnts = _part_sc(srcpad, dstpad)
    so4 = so.reshape(NC, NS, CHA, KA)
    do4 = do_.reshape(NC, NS, CHA, KA)

    m0 = _tc_k0(x, W0)
    degp = _deg_sc(dstd)
    y0, dinv = _tc_k1(degp[0, :N, :], degp[1, :N, :], m0)

    parts0 = _agg_sc(y0, so4, do4, cnts)
    y1 = _tc_k2(parts0, y0, dinv, b0.reshape(1, D), W1)

    parts1 = _agg_sc(y1, so4, do4, cnts)
    z, sums = _tc_k3(parts1, y1, dinv, b1.reshape(1, D), Wf1, bf1.reshape(1, D))

    mu = sums[0:1, :] / N
    var = sums[1:2, :] / N - mu * mu
    scale = gf.reshape(1, D) * lax.rsqrt(var + 1e-5)
    shift = betaf.reshape(1, D) - mu * scale

    pmax = _tc_k4(z, scale, shift, Wf2, bf2.reshape(1, D), batch.reshape(N, 1))

    wm2p = jnp.pad(Wm2, ((0, 0), (0, D - 1)))
    bm2p = jnp.pad(bm2.reshape(1, 1), ((0, 0), (0, D - 1)))
    out = _tc_k5(pmax, Wm1, bm1.reshape(1, D // 2), gm.reshape(1, D // 2),
                 betam.reshape(1, D // 2), wm2p, bm2p)
    return out[:, 0]


# speedup vs baseline: 1.2459x; 1.0542x over previous
import functools

import jax
import jax.numpy as jnp
from jax import lax
from jax.experimental import pallas as pl
from jax.experimental.pallas import tpu as pltpu
from jax.experimental.pallas import tpu_sc as plsc

N = 10000
E = 320000
D = 128
G = 32
NC, NS, L = 2, 16, 16
K = 128
EPT = 20480
CHUNKS = EPT // K
DCHUNKS = CHUNKS // NC
EPAD = NS * EPT
NPAD = 10240
RPT = NPAD // NS
PADDST = N
KA = 64
CHA = EPT // KA
HALFN = N // 2
AROWS = 5120
RPT2 = AROWS // NS
TRASH = HALFN
VPE = EPT // L

_mesh = plsc.VectorSubcoreMesh(core_axis_name="c", subcore_axis_name="s")
_f32 = jnp.float32
_i32 = jnp.int32


@functools.partial(
    pl.kernel,
    out_type=jax.ShapeDtypeStruct((NC, NPAD, L), _f32),
    mesh=_mesh,
    scratch_types=[
        pltpu.VMEM((DCHUNKS, K), _i32),
        pltpu.VMEM((K, L), _f32),
        pltpu.VMEM((K, L), _f32),
        pltpu.VMEM_SHARED((NPAD, L), _f32),
    ],
    compiler_params=pltpu.CompilerParams(use_tc_tiling_on_sc=False),
)
def _deg_sc(dst_hbm, out_hbm, didx, oh_buf, zbuf, acc):
    c = lax.axis_index("c")
    s = lax.axis_index("s")
    oh = jnp.where(lax.iota(_i32, L) == 0, _f32(1.0), _f32(0.0))
    z16 = jnp.zeros((L,), _f32)

    @pl.loop(0, K)
    def _(j):
        oh_buf[j, pl.ds(0, L)] = oh
        zbuf[j, pl.ds(0, L)] = z16

    @pl.loop(0, RPT // K)
    def _(j):
        pltpu.sync_copy(zbuf, acc.at[pl.ds(s * RPT + j * K, K)])

    plsc.subcore_barrier()
    pltpu.sync_copy(dst_hbm.at[c, s], didx)

    @pl.loop(0, DCHUNKS)
    def _(j):
        pltpu.sync_copy(oh_buf, acc.at[didx.at[j]], add=True)

    plsc.subcore_barrier()
    pltpu.sync_copy(acc.at[pl.ds(s * RPT, RPT)], out_hbm.at[c, pl.ds(s * RPT, RPT)])


@functools.partial(
    pl.kernel,
    out_type=[jax.ShapeDtypeStruct((NC, NS, EPT), _i32),
              jax.ShapeDtypeStruct((NC, NS, EPT), _i32),
              jax.ShapeDtypeStruct((NC, NS, L), _i32)],
    mesh=_mesh,
    scratch_types=[
        pltpu.VMEM((EPT,), _i32),
        pltpu.VMEM((EPT,), _i32),
        pltpu.VMEM((EPT,), _i32),
        pltpu.VMEM((EPT,), _i32),
        pltpu.VMEM((L,), _i32),
    ],
    compiler_params=pltpu.CompilerParams(use_tc_tiling_on_sc=False,
                                         needs_layout_passes=False),
)
def _part_sc(src_hbm, dst_hbm, so_hbm, do_hbm, cnt_hbm,
             sbuf, dbuf, sobuf, dobuf, cbuf):
    c = lax.axis_index("c")
    s = lax.axis_index("s")
    lo = c * HALFN
    pltpu.sync_copy(src_hbm.at[s], sbuf)
    pltpu.sync_copy(dst_hbm.at[s], dbuf)
    trash_s = jnp.zeros((L,), _i32)
    trash_d = jnp.full((L,), TRASH, _i32)

    @pl.loop(0, VPE)
    def _(v):
        sobuf[pl.ds(v * L, L)] = trash_s
        dobuf[pl.ds(v * L, L)] = trash_d

    @pl.loop(0, VPE, init_carry=jnp.int32(0))
    def cnt(v, cur):
        s16 = sbuf[pl.ds(v * L, L)]
        d16 = dbuf[pl.ds(v * L, L)]
        dl = d16 - lo
        m = (dl >= 0) & (dl < HALFN)
        plsc.store_compressed(sobuf.at[pl.ds(cur, L)], s16, mask=m)
        plsc.store_compressed(dobuf.at[pl.ds(cur, L)], dl, mask=m)
        return cur + jnp.sum(jnp.where(m, 1, 0).astype(_i32))

    pltpu.sync_copy(sobuf, so_hbm.at[c, s])
    pltpu.sync_copy(dobuf, do_hbm.at[c, s])
    cbuf[pl.ds(0, L)] = jnp.full((L,), cnt, _i32)
    pltpu.sync_copy(cbuf, cnt_hbm.at[c, s])


_NBUF = 4


@functools.partial(
    pl.kernel,
    out_type=jax.ShapeDtypeStruct((NC, AROWS, D), _f32),
    mesh=_mesh,
    scratch_types=[
        pltpu.VMEM((CHA, KA), _i32),
        pltpu.VMEM((CHA, KA), _i32),
        [pltpu.VMEM((KA, D), _f32)] * _NBUF,
        pltpu.VMEM((L,), _i32),
        pltpu.VMEM_SHARED((AROWS, D), _f32),
        [pltpu.SemaphoreType.DMA] * _NBUF,
        [pltpu.SemaphoreType.DMA] * _NBUF,
    ],
    compiler_params=pltpu.CompilerParams(use_tc_tiling_on_sc=False,
                                         needs_layout_passes=False),
)
def _agg_sc(y_hbm, src_hbm, dst_hbm, cnt_hbm, out_hbm,
            sidx, didx, rows, cbuf, acc, gsem, ssem):
    c = lax.axis_index("c")
    s = lax.axis_index("s")
    z16 = jnp.zeros((L,), _f32)

    zbuf = rows[0]

    @pl.loop(0, KA)
    def _(j):
        @pl.loop(0, D, step=L)
        def _(k2):
            zbuf[j, pl.ds(k2, L)] = z16

    @pl.loop(0, RPT2 // KA)
    def _(j):
        pltpu.sync_copy(zbuf, acc.at[pl.ds(s * RPT2 + j * KA, KA)])

    plsc.subcore_barrier()
    pltpu.sync_copy(src_hbm.at[c, s], sidx)
    pltpu.sync_copy(dst_hbm.at[c, s], didx)
    pltpu.sync_copy(cnt_hbm.at[c, s], cbuf)
    cnt = jnp.max(cbuf[...])
    nch = (cnt + (KA - 1)) // KA
    nrounds = jnp.maximum((nch + (_NBUF - 1)) // _NBUF, 1)

    def g_start(j, b):
        pltpu.async_copy(y_hbm.at[sidx.at[j]], rows[b], gsem[b])

    def g_wait(j, b):
        pltpu.make_async_copy(y_hbm.at[sidx.at[j]], rows[b], gsem[b]).wait()

    def s_start(j, b):
        pltpu.async_copy(rows[b], acc.at[didx.at[j]], ssem[b], add=True)

    def s_wait(j, b):
        pltpu.make_async_copy(rows[b], acc.at[didx.at[j]], ssem[b]).wait()

    for b in range(_NBUF):
        g_start(b, b)

    @pl.loop(0, nrounds - 1)
    def _(i):
        j0 = i * _NBUF
        for b in range(_NBUF):
            g_wait(j0 + b, b)
            s_start(j0 + b, b)
        for b in range(_NBUF):
            s_wait(j0 + b, b)
            g_start(j0 + _NBUF + b, b)

    j0 = (nrounds - 1) * _NBUF
    for b in range(_NBUF):
        g_wait(j0 + b, b)
        s_start(j0 + b, b)
    for b in range(_NBUF):
        s_wait(j0 + b, b)

    plsc.subcore_barrier()
    pltpu.sync_copy(acc.at[pl.ds(s * RPT2, RPT2)],
                    out_hbm.at[c, pl.ds(s * RPT2, RPT2)])


_BLK = 1000
_GRID = N // _BLK
_PB = HALFN // _BLK


def _row_spec(w):
    return pl.BlockSpec((_BLK, w), lambda i: (i, 0))


def _part_spec():
    return pl.BlockSpec((1, _BLK, D), lambda i: (i // _PB, i % _PB, 0))


def _full_spec(h, w):
    return pl.BlockSpec((h, w), lambda i: (0, 0))


def _k0_body(x, w0, m0):
    m0[...] = jnp.dot(x[...], w0[...], preferred_element_type=_f32)


def _tc_k0(x, w0):
    return pl.pallas_call(
        _k0_body,
        grid=(_GRID,),
        in_specs=[_row_spec(D), _full_spec(D, D)],
        out_specs=_row_spec(D),
        out_shape=jax.ShapeDtypeStruct((N, D), _f32),
    )(x, w0)


def _k1_body(dg0, dg1, m0, y0, dinv):
    deg = dg0[:, 0:1] + dg1[:, 0:1] + _f32(1.0)
    di = lax.rsqrt(jnp.maximum(deg, _f32(1.0)))
    dinv[...] = di
    y0[...] = m0[...] * di


def _tc_k1(dg0, dg1, m0):
    return pl.pallas_call(
        _k1_body,
        grid=(_GRID,),
        in_specs=[_row_spec(L), _row_spec(L), _row_spec(D)],
        out_specs=[_row_spec(D), _row_spec(1)],
        out_shape=[jax.ShapeDtypeStruct((N, D), _f32),
                   jax.ShapeDtypeStruct((N, 1), _f32)],
    )(dg0, dg1, m0)


def _k2_body(p, y, dinv, b, wn, out):
    h = jax.nn.relu((p[0] + y[...]) * dinv[...] + b[...])
    out[...] = jnp.dot(h, wn[...], preferred_element_type=_f32) * dinv[...]


def _tc_k2(p, y, dinv, b, wn):
    return pl.pallas_call(
        _k2_body,
        grid=(_GRID,),
        in_specs=[_part_spec(), _row_spec(D), _row_spec(1),
                  _full_spec(1, D), _full_spec(D, D)],
        out_specs=_row_spec(D),
        out_shape=jax.ShapeDtypeStruct((N, D), _f32),
    )(p, y, dinv, b, wn)


def _k3_body(p, y, dinv, b, wf1, bf1, z, sums):
    h = jax.nn.relu((p[0] + y[...]) * dinv[...] + b[...])
    zz = jnp.dot(h, wf1[...], preferred_element_type=_f32) + bf1[...]
    z[...] = zz

    @pl.when(pl.program_id(0) == 0)
    def _():
        sums[...] = jnp.zeros_like(sums)

    sums[0:1, :] += jnp.sum(zz, axis=0, keepdims=True)
    sums[1:2, :] += jnp.sum(zz * zz, axis=0, keepdims=True)


def _tc_k3(p, y, dinv, b, wf1, bf1):
    return pl.pallas_call(
        _k3_body,
        grid=(_GRID,),
        in_specs=[_part_spec(), _row_spec(D), _row_spec(1),
                  _full_spec(1, D), _full_spec(D, D), _full_spec(1, D)],
        out_specs=[_row_spec(D), _full_spec(8, D)],
        out_shape=[jax.ShapeDtypeStruct((N, D), _f32),
                   jax.ShapeDtypeStruct((8, D), _f32)],
    )(p, y, dinv, b, wf1, bf1)


def _k4_body(z, scale, shift, wf2, bf2, bat, pmax):
    zn = jax.nn.relu(z[...] * scale[...] + shift[...])
    h3 = jax.nn.relu(jnp.dot(zn, wf2[...], preferred_element_type=_f32) + bf2[...])
    bb = bat[...]

    @pl.when(pl.program_id(0) == 0)
    def _():
        pmax[...] = jnp.full_like(pmax, -jnp.inf)

    def _one_graph(g, carry):
        m = jnp.max(jnp.where(bb == g, h3, -jnp.inf), axis=0, keepdims=True)
        pmax[pl.ds(g, 1), :] = jnp.maximum(pmax[pl.ds(g, 1), :], m)
        return carry

    lax.fori_loop(bb[0, 0], bb[_BLK - 1, 0] + 1, _one_graph, 0)


def _tc_k4(z, scale, shift, wf2, bf2, bat):
    return pl.pallas_call(
        _k4_body,
        grid=(_GRID,),
        in_specs=[_row_spec(D), _full_spec(1, D), _full_spec(1, D),
                  _full_spec(D, D), _full_spec(1, D), _row_spec(1)],
        out_specs=_full_spec(G, D),
        out_shape=jax.ShapeDtypeStruct((G, D), _f32),
    )(z, scale, shift, wf2, bf2, bat)


def _k5_body(p, wm1, bm1, gm, betam, wm2, bm2, out):
    p1 = jnp.dot(p[...], wm1[...], preferred_element_type=_f32) + bm1[...]
    mu = jnp.mean(p1, axis=0, keepdims=True)
    var = jnp.mean((p1 - mu) * (p1 - mu), axis=0, keepdims=True)
    p1 = (p1 - mu) * lax.rsqrt(var + _f32(1e-5)) * gm[...] + betam[...]
    p1 = jax.nn.relu(p1)
    out[...] = jnp.dot(p1, wm2[...], preferred_element_type=_f32) + bm2[...]


def _tc_k5(p, wm1, bm1, gm, betam, wm2, bm2):
    HW = D // 2
    return pl.pallas_call(
        _k5_body,
        grid=(1,),
        in_specs=[_full_spec(G, D), _full_spec(D, HW), _full_spec(1, HW),
                  _full_spec(1, HW), _full_spec(1, HW), _full_spec(HW, D),
                  _full_spec(1, D)],
        out_specs=_full_spec(G, D),
        out_shape=jax.ShapeDtypeStruct((G, D), _f32),
    )(p, wm1, bm1, gm, betam, wm2, bm2)


def kernel(x, edge_index, edge_attr, batch, W0, b0, W1, b1, Wf1, bf1, gf, betaf,
           Wf2, bf2, Wm1, bm1, gm, betam, Wm2, bm2):
    src = edge_index[0]
    dst = edge_index[1]
    srcpad = jnp.concatenate([src, jnp.zeros((EPAD - E,), _i32)]).reshape(NS, EPT)
    dstpad = jnp.concatenate([dst, jnp.full((EPAD - E,), PADDST, _i32)]
                             ).reshape(NS, EPT)
    dstd = dstpad.reshape(NS, NC, DCHUNKS, K).transpose(1, 0, 2, 3)

    so, do_, cnts = _part_sc(srcpad, dstpad)
    so4 = so.reshape(NC, NS, CHA, KA)
    do4 = do_.reshape(NC, NS, CHA, KA)

    m0 = _tc_k0(x, W0)
    degp = _deg_sc(dstd)
    y0, dinv = _tc_k1(degp[0, :N, :], degp[1, :N, :], m0)

    parts0 = _agg_sc(y0, so4, do4, cnts)
    y1 = _tc_k2(parts0, y0, dinv, b0.reshape(1, D), W1)

    parts1 = _agg_sc(y1, so4, do4, cnts)
    z, sums = _tc_k3(parts1, y1, dinv, b1.reshape(1, D), Wf1, bf1.reshape(1, D))

    mu = sums[0:1, :] / N
    var = sums[1:2, :] / N - mu * mu
    scale = gf.reshape(1, D) * lax.rsqrt(var + 1e-5)
    shift = betaf.reshape(1, D) - mu * scale

    pmax = _tc_k4(z, scale, shift, Wf2, bf2.reshape(1, D), batch.reshape(N, 1))

    wm2p = jnp.pad(Wm2, ((0, 0), (0, D - 1)))
    bm2p = jnp.pad(bm2.reshape(1, 1), ((0, 0), (0, D - 1)))
    out = _tc_k5(pmax, Wm1, bm1.reshape(1, D // 2), gm.reshape(1, D // 2),
                 betam.reshape(1, D // 2), wm2p, bm2p)
    return out[:, 0]
